# baseline (device time: 8094 ns/iter reference)
import jax
import jax.numpy as jnp
from jax import lax
from jax.experimental import pallas as pl
from jax.experimental.pallas import tpu as pltpu


def kernel(x):
    _, m, n = x.shape
    half = n // 2

    def body(x_hbm, out_ref, xv_ref, comm_ref, res_ref, copy_sem, out_sem,
             send_sem, recv_sem):
        my_x = lax.axis_index("x")
        my_y = lax.axis_index("y")
        my_z = lax.axis_index("z")
        peer = (my_x, 1 - my_y, my_z)

        cp = pltpu.make_async_copy(x_hbm.at[0], xv_ref, copy_sem)
        cp.start()

        barrier_sem = pltpu.get_barrier_semaphore()
        pl.semaphore_signal(
            barrier_sem, inc=1,
            device_id=peer, device_id_type=pl.DeviceIdType.MESH,
        )
        pl.semaphore_wait(barrier_sem, 1)
        cp.wait()

        def exchange(send_lo, keep_lo):
            rdma = pltpu.make_async_remote_copy(
                src_ref=xv_ref.at[:, pl.ds(send_lo, half)],
                dst_ref=comm_ref,
                send_sem=send_sem,
                recv_sem=recv_sem,
                device_id=peer,
                device_id_type=pl.DeviceIdType.MESH,
            )
            rdma.start()
            rdma.wait()
            res_ref[:, :] = xv_ref[:, pl.ds(keep_lo, half)] + comm_ref[:, :]

        @pl.when(my_y == 0)
        def _():
            exchange(half, 0)

        @pl.when(my_y == 1)
        def _():
            exchange(0, half)

        outcp = pltpu.make_async_copy(res_ref, out_ref, out_sem)
        outcp.start()
        outcp.wait()

    return pl.pallas_call(
        body,
        out_shape=jax.ShapeDtypeStruct((m, half), x.dtype),
        in_specs=[pl.BlockSpec(memory_space=pl.ANY)],
        out_specs=pl.BlockSpec(memory_space=pl.ANY),
        scratch_shapes=[
            pltpu.VMEM((m, n), x.dtype),
            pltpu.VMEM((m, half), x.dtype),
            pltpu.VMEM((m, half), x.dtype),
            pltpu.SemaphoreType.DMA,
            pltpu.SemaphoreType.DMA,
            pltpu.SemaphoreType.DMA,
            pltpu.SemaphoreType.DMA,
        ],
        compiler_params=pltpu.CompilerParams(collective_id=0),
    )(x)


# device time: 8033 ns/iter; 1.0076x vs baseline; 1.0076x over previous
import jax
import jax.numpy as jnp
from jax import lax
from jax.experimental import pallas as pl
from jax.experimental.pallas import tpu as pltpu


def kernel(x):
    _, m, n = x.shape
    half = n // 2

    def body(x_hbm, out_ref, xv_ref, comm_ref, res_ref, copy_sem, out_sem,
             send_sem, recv_sem):
        my_x = lax.axis_index("x")
        my_y = lax.axis_index("y")
        my_z = lax.axis_index("z")
        peer = (my_x, 1 - my_y, my_z)

        cp = pltpu.make_async_copy(x_hbm.at[0], xv_ref, copy_sem)
        cp.start()

        barrier_sem = pltpu.get_barrier_semaphore()
        pl.semaphore_signal(
            barrier_sem, inc=1,
            device_id=peer, device_id_type=pl.DeviceIdType.MESH,
        )
        pl.semaphore_wait(barrier_sem, 1)
        cp.wait()

        def exchange(send_lo, keep_lo):
            rdma = pltpu.make_async_remote_copy(
                src_ref=xv_ref.at[:, pl.ds(send_lo, half)],
                dst_ref=comm_ref,
                send_sem=send_sem,
                recv_sem=recv_sem,
                device_id=peer,
                device_id_type=pl.DeviceIdType.MESH,
            )
            rdma.start()
            rdma.wait()
            res_ref[:, :] = xv_ref[:, pl.ds(keep_lo, half)] + comm_ref[:, :]

        @pl.when(my_y == 0)
        def _():
            exchange(half, 0)

        @pl.when(my_y == 1)
        def _():
            exchange(0, half)

        outcp = pltpu.make_async_copy(res_ref, out_ref, out_sem)
        outcp.start()
        outcp.wait()

    x = pltpu.with_memory_space_constraint(x, pltpu.MemorySpace.HBM)
    return pl.pallas_call(
        body,
        out_shape=jax.ShapeDtypeStruct((m, half), x.dtype),
        in_specs=[pl.BlockSpec(memory_space=pltpu.MemorySpace.HBM)],
        out_specs=pl.BlockSpec(memory_space=pltpu.MemorySpace.HBM),
        scratch_shapes=[
            pltpu.VMEM((m, n), x.dtype),
            pltpu.VMEM((m, half), x.dtype),
            pltpu.VMEM((m, half), x.dtype),
            pltpu.SemaphoreType.DMA,
            pltpu.SemaphoreType.DMA,
            pltpu.SemaphoreType.DMA,
            pltpu.SemaphoreType.DMA,
        ],
        compiler_params=pltpu.CompilerParams(collective_id=0),
    )(x)


# device time: 7997 ns/iter; 1.0121x vs baseline; 1.0045x over previous
import jax
import jax.numpy as jnp
from jax import lax
from jax.experimental import pallas as pl
from jax.experimental.pallas import tpu as pltpu

N_CHUNKS = 2


def kernel(x):
    _, m, n = x.shape
    half = n // 2
    rows = m // N_CHUNKS

    def body(x_hbm, out_ref, xv_ref, comm_ref, copy_sem, send_sems, recv_sems):
        my_x = lax.axis_index("x")
        my_y = lax.axis_index("y")
        my_z = lax.axis_index("z")
        peer = (my_x, 1 - my_y, my_z)

        cp = pltpu.make_async_copy(x_hbm.at[0], xv_ref, copy_sem)
        cp.start()

        barrier_sem = pltpu.get_barrier_semaphore()
        pl.semaphore_signal(
            barrier_sem, inc=1,
            device_id=peer, device_id_type=pl.DeviceIdType.MESH,
        )
        pl.semaphore_wait(barrier_sem, 1)
        cp.wait()

        def exchange(send_lo, keep_lo):
            rdmas = []
            for c in range(N_CHUNKS):
                rdma = pltpu.make_async_remote_copy(
                    src_ref=xv_ref.at[pl.ds(c * rows, rows),
                                      pl.ds(send_lo, half)],
                    dst_ref=comm_ref.at[c],
                    send_sem=send_sems.at[c],
                    recv_sem=recv_sems.at[c],
                    device_id=peer,
                    device_id_type=pl.DeviceIdType.MESH,
                )
                rdma.start()
                rdmas.append(rdma)
            for c in range(N_CHUNKS):
                rdmas[c].wait_recv()
                out_ref[pl.ds(c * rows, rows), :] = (
                    xv_ref[pl.ds(c * rows, rows), pl.ds(keep_lo, half)]
                    + comm_ref[c]
                )
            for c in range(N_CHUNKS):
                rdmas[c].wait_send()

        @pl.when(my_y == 0)
        def _():
            exchange(half, 0)

        @pl.when(my_y == 1)
        def _():
            exchange(0, half)

    x = pltpu.with_memory_space_constraint(x, pltpu.MemorySpace.HBM)
    return pl.pallas_call(
        body,
        out_shape=jax.ShapeDtypeStruct((m, half), x.dtype),
        in_specs=[pl.BlockSpec(memory_space=pltpu.MemorySpace.HBM)],
        out_specs=pl.BlockSpec(memory_space=pltpu.VMEM),
        scratch_shapes=[
            pltpu.VMEM((m, n), x.dtype),
            pltpu.VMEM((N_CHUNKS, rows, half), x.dtype),
            pltpu.SemaphoreType.DMA,
            pltpu.SemaphoreType.DMA((N_CHUNKS,)),
            pltpu.SemaphoreType.DMA((N_CHUNKS,)),
        ],
        compiler_params=pltpu.CompilerParams(collective_id=0),
    )(x)
